# baseline (speedup 1.0000x reference)
import jax
import jax.numpy as jnp
from jax.experimental import pallas as pl
from jax.experimental.pallas import tpu as pltpu

_BATCH = 16384
_DIM = 256
_NPROBES = 8192
_TEMP = 0.2
_ALPHA = 0.5
_GATE_THRESHOLD = 0.1
_TB = 256


def _body(x_ref, p_ref, s0_ref, win_ref, conf_ref, mraw_ref, probs_ref, raw_ref):
    p = p_ref[...]
    x = x_ref[...]
    nrm = jnp.sqrt(jnp.sum(x * x, axis=1, keepdims=True))
    xn = x / jnp.maximum(nrm, 1e-12)
    raw = jax.lax.dot_general(
        xn, p, (((1,), (1,)), ((), ())), preferred_element_type=jnp.float32
    )
    raw_ref[...] = raw
    e = jnp.exp(raw * (1.0 / _TEMP))
    w = jax.lax.dot_general(
        e, p, (((1,), (0,)), ((), ())), preferred_element_type=jnp.float32
    )
    s = jnp.sum(e, axis=1, keepdims=True)
    re = jnp.sum(raw * e, axis=1, keepdims=True)
    mr = jnp.max(raw, axis=1, keepdims=True)
    rs = 1.0 / s
    probs_ref[...] = e * rs
    avg = re * rs
    gate = jax.nn.sigmoid((avg - _GATE_THRESHOLD) * 10.0)
    s0_ref[...] = (_ALPHA * x + (1.0 - _ALPHA) * (w * rs)) * gate
    win_ref[...] = jnp.argmax(raw, axis=1, keepdims=True).astype(jnp.int32)
    conf_ref[...] = jnp.exp(mr * (1.0 / _TEMP)) * rs
    mraw_ref[...] = mr


def kernel(x_input, probes):
    nblocks = _BATCH // _TB
    out_shapes = (
        jax.ShapeDtypeStruct((_BATCH, _DIM), jnp.float32),
        jax.ShapeDtypeStruct((_BATCH, 1), jnp.int32),
        jax.ShapeDtypeStruct((_BATCH, 1), jnp.float32),
        jax.ShapeDtypeStruct((_BATCH, 1), jnp.float32),
        jax.ShapeDtypeStruct((_BATCH, _NPROBES), jnp.float32),
        jax.ShapeDtypeStruct((_BATCH, _NPROBES), jnp.float32),
    )
    s0, win, conf, mraw, probs_o, raw_o = pl.pallas_call(
        _body,
        grid=(nblocks,),
        in_specs=[
            pl.BlockSpec((_TB, _DIM), lambda i: (i, 0)),
            pl.BlockSpec((_NPROBES, _DIM), lambda i: (0, 0)),
        ],
        out_specs=(
            pl.BlockSpec((_TB, _DIM), lambda i: (i, 0)),
            pl.BlockSpec((_TB, 1), lambda i: (i, 0)),
            pl.BlockSpec((_TB, 1), lambda i: (i, 0)),
            pl.BlockSpec((_TB, 1), lambda i: (i, 0)),
            pl.BlockSpec((_TB, _NPROBES), lambda i: (i, 0)),
            pl.BlockSpec((_TB, _NPROBES), lambda i: (i, 0)),
        ),
        out_shape=out_shapes,
        compiler_params=pltpu.CompilerParams(
            dimension_semantics=("parallel",),
        ),
    )(x_input, probes)
    win = win[:, 0]
    conf = conf[:, 0]
    mraw = mraw[:, 0]
    gate_open = mraw > _GATE_THRESHOLD
    return (s0, win, conf, mraw, gate_open, probs_o, raw_o)
